# baseline (speedup 1.0000x reference)
import functools

import jax
import jax.numpy as jnp
from jax import lax
from jax.experimental import pallas as pl
from jax.experimental.pallas import tpu as pltpu
from jax.experimental.pallas import tpu_sc as plsc

VOCAB = 21128
HIDDEN = 768
MAX_POS = 512
B = 128
S = 512
LN_EPS = 1e-12

NW = 32
SEQ_PER_W = B // NW
SCHUNK = 64
NSC = S // SCHUNK
NJ = HIDDEN // 16
INV_H = 1.0 / HIDDEN


def _rsqrt(v):
    i = lax.bitcast_convert_type(v, jnp.int32)
    i = jnp.int32(0x5F3759DF) - lax.shift_right_arithmetic(i, jnp.int32(1))
    y = lax.bitcast_convert_type(i, jnp.float32)
    half = v * 0.5
    for _ in range(3):
        y = y * (1.5 - half * y * y)
    return y


def _body(ids_hbm, tts_hbm, word_hbm, pos_hbm, type_hbm, gam_hbm, bet_hbm,
          out_hbm, pos_v, rows_v, type_v, gam_v, bet_v, idx_v, tt_v, gsem):
    cid = lax.axis_index("c")
    sid = lax.axis_index("s")
    wid = sid * 2 + cid
    seq0 = wid * SEQ_PER_W

    pltpu.sync_copy(type_hbm, type_v)
    pltpu.sync_copy(gam_hbm, gam_v)
    pltpu.sync_copy(bet_hbm, bet_v)

    def schunk_body(sc, _):
        s0 = sc * SCHUNK
        pltpu.sync_copy(pos_hbm.at[pl.ds(s0, SCHUNK)], pos_v)

        def seq_body(b, _):
            tokbase = (seq0 + b) * S + s0
            pltpu.sync_copy(ids_hbm.at[pl.ds(tokbase, SCHUNK)], idx_v)
            pltpu.sync_copy(tts_hbm.at[pl.ds(tokbase, SCHUNK)],
                            tt_v.at[pl.ds(0, SCHUNK)])
            pltpu.async_copy(word_hbm.at[idx_v], rows_v, gsem).wait()

            def tok_body(t, _):
                ttf = jnp.full(
                    (16,), tt_v[pl.ds(t, 16)][0].astype(jnp.float32))
                sumv = jnp.zeros((16,), jnp.float32)
                sqv = jnp.zeros((16,), jnp.float32)
                for j in range(NJ):
                    d = pl.ds(j * 16, 16)
                    x = (rows_v[t, d] + pos_v[t, d]
                         + type_v[0, d] + ttf * (type_v[1, d] - type_v[0, d]))
                    rows_v[t, d] = x
                    sumv = sumv + x
                    sqv = sqv + x * x
                mean = jnp.sum(sumv) * INV_H
                var = jnp.sum(sqv) * INV_H - mean * mean
                rstd = _rsqrt(jnp.full((16,), var + LN_EPS, jnp.float32))
                mvec = jnp.full((16,), mean, jnp.float32)
                for j in range(NJ):
                    d = pl.ds(j * 16, 16)
                    rows_v[t, d] = ((rows_v[t, d] - mvec) * rstd * gam_v[d]
                                    + bet_v[d])
                return 0

            lax.fori_loop(0, SCHUNK, tok_body, 0)
            pltpu.sync_copy(rows_v, out_hbm.at[pl.ds(tokbase, SCHUNK)])
            return 0

        lax.fori_loop(0, SEQ_PER_W, seq_body, 0)
        return 0

    lax.fori_loop(0, NSC, schunk_body, 0)


_sc_call = pl.kernel(
    _body,
    out_type=jax.ShapeDtypeStruct((B * S, HIDDEN), jnp.float32),
    mesh=plsc.VectorSubcoreMesh(core_axis_name="c", subcore_axis_name="s"),
    compiler_params=pltpu.CompilerParams(needs_layout_passes=False),
    scratch_types=[
        pltpu.VMEM((SCHUNK, HIDDEN), jnp.float32),
        pltpu.VMEM((SCHUNK, HIDDEN), jnp.float32),
        pltpu.VMEM((2, HIDDEN), jnp.float32),
        pltpu.VMEM((HIDDEN,), jnp.float32),
        pltpu.VMEM((HIDDEN,), jnp.float32),
        pltpu.VMEM((SCHUNK,), jnp.int32),
        pltpu.VMEM((SCHUNK + 16,), jnp.int32),
        pltpu.SemaphoreType.DMA,
    ],
)


@jax.jit
def kernel(input_ids, token_type_ids, word_embeddings, position_embeddings,
           token_type_embeddings, ln_gamma, ln_beta):
    ids = input_ids.reshape(-1).astype(jnp.int32)
    tts = token_type_ids.reshape(-1).astype(jnp.int32)
    out = _sc_call(ids, tts, word_embeddings, position_embeddings,
                   token_type_embeddings, ln_gamma, ln_beta)
    return out.reshape(input_ids.shape[0], input_ids.shape[1], HIDDEN)
